# initial kernel scaffold (unmeasured)
import jax
import jax.numpy as jnp
from jax import lax
from jax.experimental import pallas as pl
from jax.experimental.pallas import tpu as pltpu

N_DEV = 16


def kernel(x, router_W, route_idx, expert_W, shared_W):
    n_tokens, d_model = x.shape
    n_local_e, _, d_hidden = expert_W.shape
    m_per = n_tokens // N_DEV

    def body(x_ref, rw_ref, idx_ref, ew_ref, sw_ref, out_ref,
             partial_ref, gather_ref, send_sems, recv_sems):
        my = lax.axis_index("i")

        scores = jnp.dot(x_ref[:], rw_ref[:],
                         preferred_element_type=jnp.float32)
        s_max = jnp.max(scores, axis=1, keepdims=True)
        p = jnp.exp(scores - s_max)
        probs = p / jnp.sum(p, axis=1, keepdims=True)
        route = idx_ref[:]
        iota_e = lax.broadcasted_iota(jnp.int32, scores.shape, 1)
        prob_sel = jnp.sum(jnp.where(route == iota_e, probs, 0.0),
                           axis=1, keepdims=True)

        partial = jnp.zeros((n_tokens, d_hidden), dtype=jnp.float32)
        for le in range(n_local_e):
            e = my * n_local_e + le
            coeff = jnp.where(route == e, prob_sel, 0.0)
            partial = partial + jnp.dot(
                x_ref[:] * coeff, ew_ref[le],
                preferred_element_type=jnp.float32)
        partial_ref[:] = partial

        gather_ref[0] = partial_ref[pl.ds(my * m_per, m_per), :]

        descs = []
        for k in range(1, N_DEV):
            dst = (my + k) % N_DEV
            desc = pltpu.make_async_remote_copy(
                src_ref=partial_ref.at[pl.ds(dst * m_per, m_per)],
                dst_ref=gather_ref.at[k],
                send_sem=send_sems.at[k],
                recv_sem=recv_sems.at[k],
                device_id=(dst,),
                device_id_type=pl.DeviceIdType.MESH,
            )
            desc.start()
            descs.append(desc)
        for desc in descs:
            desc.wait_recv()
        for desc in descs:
            desc.wait_send()

        shared_blk = jnp.dot(x_ref[pl.ds(my * m_per, m_per), :], sw_ref[:],
                             preferred_element_type=jnp.float32)
        out_ref[:] = shared_blk + jnp.sum(gather_ref[:], axis=0)

    return pl.pallas_call(
        body,
        out_shape=jax.ShapeDtypeStruct((m_per, d_hidden), jnp.float32),
        in_specs=[pl.BlockSpec(memory_space=pltpu.VMEM)] * 5,
        out_specs=pl.BlockSpec(memory_space=pltpu.VMEM),
        scratch_shapes=[
            pltpu.VMEM((n_tokens, d_hidden), jnp.float32),
            pltpu.VMEM((N_DEV, m_per, d_hidden), jnp.float32),
            pltpu.SemaphoreType.DMA((N_DEV,)),
            pltpu.SemaphoreType.DMA((N_DEV,)),
        ],
        compiler_params=pltpu.CompilerParams(collective_id=0),
    )(x, router_W, route_idx, expert_W, shared_W)


# baseline (device time: 27663 ns/iter reference)
import jax
import jax.numpy as jnp
from jax import lax
from jax.experimental import pallas as pl
from jax.experimental.pallas import tpu as pltpu

N_DEV = 16


def kernel(x, router_W, route_idx, expert_W, shared_W):
    n_tokens, d_model = x.shape
    n_local_e, _, d_hidden = expert_W.shape
    m_per = n_tokens // N_DEV

    def body(x_ref, rw_ref, idx_ref, ew_ref, sw_ref, out_ref,
             partial_ref, gather_ref, send_sems, recv_sems):
        my = lax.axis_index("i")

        scores = jnp.dot(x_ref[:], rw_ref[:],
                         preferred_element_type=jnp.float32)
        s_max = jnp.max(scores, axis=1, keepdims=True)
        p = jnp.exp(scores - s_max)
        probs = p / jnp.sum(p, axis=1, keepdims=True)
        route = idx_ref[:]
        iota_e = lax.broadcasted_iota(jnp.int32, scores.shape, 1)
        prob_sel = jnp.sum(jnp.where(route == iota_e, probs, 0.0),
                           axis=1, keepdims=True)

        partial = jnp.zeros((n_tokens, d_hidden), dtype=jnp.float32)
        for le in range(n_local_e):
            e = my * n_local_e + le
            coeff = jnp.where(route == e, prob_sel, 0.0)
            partial = partial + jnp.dot(
                x_ref[:] * coeff, ew_ref[le],
                preferred_element_type=jnp.float32)
        partial_ref[:] = partial

        gather_ref[0] = partial_ref[pl.ds(my * m_per, m_per), :]

        descs = []
        for k in range(1, N_DEV):
            dst = (my + k) % N_DEV
            desc = pltpu.make_async_remote_copy(
                src_ref=partial_ref.at[pl.ds(dst * m_per, m_per)],
                dst_ref=gather_ref.at[k],
                send_sem=send_sems.at[k],
                recv_sem=recv_sems.at[k],
                device_id=(dst,),
                device_id_type=pl.DeviceIdType.MESH,
            )
            desc.start()
            descs.append(desc)
        for desc in descs:
            desc.wait_recv()
        for desc in descs:
            desc.wait_send()

        shared_blk = jnp.dot(x_ref[pl.ds(my * m_per, m_per), :], sw_ref[:],
                             preferred_element_type=jnp.float32)
        out_ref[:] = shared_blk + jnp.sum(gather_ref[:], axis=0)

    return pl.pallas_call(
        body,
        out_shape=jax.ShapeDtypeStruct((m_per, d_hidden), jnp.float32),
        in_specs=[pl.BlockSpec(memory_space=pltpu.VMEM)] * 5,
        out_specs=pl.BlockSpec(memory_space=pltpu.VMEM),
        scratch_shapes=[
            pltpu.VMEM((n_tokens, d_hidden), jnp.float32),
            pltpu.VMEM((N_DEV, m_per, d_hidden), jnp.float32),
            pltpu.SemaphoreType.DMA((N_DEV,)),
            pltpu.SemaphoreType.DMA((N_DEV,)),
        ],
    )(x, router_W, route_idx, expert_W, shared_W)
